# baseline (device time: 812644 ns/iter reference)
import jax
import jax.numpy as jnp
from jax import lax
from jax.experimental import pallas as pl
from jax.experimental.pallas import tpu as pltpu

VROWS = 2048
NFSLOTS = 2
NBSLOTS = 4


def kernel(x):
    m_per, n = x.shape
    nch = m_per // VROWS

    def body(x_ref, out_ref, fbuf, bbuf, load_sems, store_sems, send_sems, recv_sems):
        my_x = lax.axis_index("x")
        my_y = lax.axis_index("y")
        my_z = lax.axis_index("z")
        nbr = (my_x, my_y, 1 - my_z)

        barrier = pltpu.get_barrier_semaphore()
        pl.semaphore_signal(
            barrier, inc=1, device_id=nbr, device_id_type=pl.DeviceIdType.MESH
        )
        pl.semaphore_wait(barrier, 1)

        base = my_z * m_per

        stores = []
        rdmas = []
        for c in range(nch):
            fslot = c % NFSLOTS
            bslot = c % NBSLOTS

            ld = pltpu.make_async_copy(
                x_ref.at[pl.ds(c * VROWS, VROWS), :],
                fbuf.at[fslot],
                load_sems.at[fslot],
            )
            ld.start()
            ld.wait()

            if c >= NBSLOTS:
                rdmas[c - NBSLOTS].wait_send()
                stores[c - NBSLOTS].wait()

            bbuf[bslot] = fbuf[fslot][...].astype(jnp.bfloat16)

            st = pltpu.make_async_copy(
                bbuf.at[bslot],
                out_ref.at[pl.ds(base + c * VROWS, VROWS), :],
                store_sems.at[bslot],
            )
            st.start()
            stores.append(st)

            r = pltpu.make_async_remote_copy(
                src_ref=bbuf.at[bslot],
                dst_ref=out_ref.at[pl.ds(base + c * VROWS, VROWS), :],
                send_sem=send_sems.at[bslot],
                recv_sem=recv_sems.at[c],
                device_id=nbr,
                device_id_type=pl.DeviceIdType.MESH,
            )
            r.start()
            rdmas.append(r)

        for c in range(nch - NBSLOTS, nch):
            rdmas[c].wait_send()
            stores[c].wait()
        for c in range(nch):
            rdmas[c].wait_recv()

    return pl.pallas_call(
        body,
        out_shape=jax.ShapeDtypeStruct((2 * m_per, n), jnp.bfloat16),
        in_specs=[pl.BlockSpec(memory_space=pl.ANY)],
        out_specs=pl.BlockSpec(memory_space=pl.ANY),
        scratch_shapes=[
            pltpu.VMEM((NFSLOTS, VROWS, n), jnp.float32),
            pltpu.VMEM((NBSLOTS, VROWS, n), jnp.bfloat16),
            pltpu.SemaphoreType.DMA((NFSLOTS,)),
            pltpu.SemaphoreType.DMA((NBSLOTS,)),
            pltpu.SemaphoreType.DMA((NBSLOTS,)),
            pltpu.SemaphoreType.DMA((nch,)),
        ],
        compiler_params=pltpu.CompilerParams(collective_id=0),
    )(x)
